# baseline (speedup 1.0000x reference)
import functools

import jax
import jax.numpy as jnp
from jax import lax
from jax.experimental import pallas as pl
from jax.experimental.pallas import tpu as pltpu
from jax.experimental.pallas import tpu_sc as plsc

_E = 16
_K = 2
_H = 1024
_I = 512
_T = 4096
_B = 128
_P = 10240
_NB = _P // _B
_NC = 2
_NS = 16
_NW = _NC * _NS
_TW = _T // _NW
_GC = 32
_NGC = _TW // _GC
_CC = 16
_NCC = _TW // _CC
_V = 16

_MESH = dict(core_axis_name="c", subcore_axis_name="s")


@functools.partial(
    pl.kernel,
    out_type=jax.ShapeDtypeStruct((_P, _H // 2), jnp.int32),
    mesh=plsc.VectorSubcoreMesh(**_MESH),
    scratch_types=[
        pltpu.VMEM((_GC,), jnp.int32),
        pltpu.VMEM((_GC,), jnp.int32),
        pltpu.VMEM((_GC,), jnp.int32),
        pltpu.VMEM((_GC,), jnp.int32),
        pltpu.VMEM((_GC, _H // 2), jnp.int32),
        pltpu.VMEM((_GC, _H // 2), jnp.int32),
        pltpu.SemaphoreType.DMA,
        pltpu.SemaphoreType.DMA,
    ],
)
def _permute_rows(src, pos0, pos1, out, i0a, i0b, i1a, i1b, bufa, bufb,
                  sem_in, sem_out):
    wid = lax.axis_index("s") * _NC + lax.axis_index("c")
    base = wid * _TW
    i0 = (i0a, i0b)
    i1 = (i1a, i1b)
    buf = (bufa, bufb)
    in_h = [None] * _NGC
    out_h = [None] * _NGC

    def load_idx(c):
        o = base + c * _GC
        pltpu.sync_copy(pos0.at[pl.ds(o, _GC)], i0[c % 2])
        pltpu.sync_copy(pos1.at[pl.ds(o, _GC)], i1[c % 2])

    load_idx(0)
    in_h[0] = pltpu.async_copy(src.at[pl.ds(base, _GC)], buf[0], sem_in)
    for c in range(_NGC):
        in_h[c].wait()
        if c + 1 < _NGC:
            if c >= 1:
                for h in out_h[c - 1]:
                    h.wait()
            load_idx(c + 1)
            o = base + (c + 1) * _GC
            in_h[c + 1] = pltpu.async_copy(
                src.at[pl.ds(o, _GC)], buf[(c + 1) % 2], sem_in)
        out_h[c] = (
            pltpu.async_copy(buf[c % 2], out.at[i0[c % 2]], sem_out),
            pltpu.async_copy(buf[c % 2], out.at[i1[c % 2]], sem_out),
        )
    for c in (_NGC - 2, _NGC - 1):
        for h in out_h[c]:
            h.wait()


@functools.partial(
    pl.kernel,
    out_type=jax.ShapeDtypeStruct((_T, _H), jnp.float32),
    mesh=plsc.VectorSubcoreMesh(**_MESH),
    scratch_types=[
        pltpu.VMEM((_CC,), jnp.int32),
        pltpu.VMEM((_CC,), jnp.int32),
        pltpu.VMEM((_CC,), jnp.int32),
        pltpu.VMEM((_CC,), jnp.int32),
        pltpu.VMEM((_CC, _V), jnp.float32),
        pltpu.VMEM((_CC, _V), jnp.float32),
        pltpu.VMEM((_CC, _V), jnp.float32),
        pltpu.VMEM((_CC, _V), jnp.float32),
        pltpu.VMEM((_CC, _H), jnp.float32),
        pltpu.VMEM((_CC, _H), jnp.float32),
        pltpu.VMEM((_CC, _H), jnp.float32),
        pltpu.VMEM((_CC, _H), jnp.float32),
        pltpu.SemaphoreType.DMA,
        pltpu.SemaphoreType.DMA,
    ],
)
def _combine_rows(y, pos0, pos1, w0m, w1m, out, i0a, i0b, i1a, i1b,
                  w0a, w0b, w1a, w1b, a0, a1, b0, b1, sem_in, sem_out):
    wid = lax.axis_index("s") * _NC + lax.axis_index("c")
    base = wid * _TW
    i0 = (i0a, i0b)
    i1 = (i1a, i1b)
    w0v = (w0a, w0b)
    w1v = (w1a, w1b)
    av = (a0, a1)
    bv = (b0, b1)
    in_h = [None] * _NCC
    out_h = [None] * _NCC

    def start_chunk(c):
        o = base + c * _CC
        pltpu.sync_copy(pos0.at[pl.ds(o, _CC)], i0[c % 2])
        pltpu.sync_copy(pos1.at[pl.ds(o, _CC)], i1[c % 2])
        pltpu.sync_copy(w0m.at[pl.ds(o, _CC)], w0v[c % 2])
        pltpu.sync_copy(w1m.at[pl.ds(o, _CC)], w1v[c % 2])
        in_h[c] = (
            pltpu.async_copy(y.at[i0[c % 2]], av[c % 2], sem_in),
            pltpu.async_copy(y.at[i1[c % 2]], bv[c % 2], sem_in),
        )

    start_chunk(0)
    for c in range(_NCC):
        for h in in_h[c]:
            h.wait()
        if c + 1 < _NCC:
            if c >= 1:
                out_h[c - 1].wait()
            start_chunk(c + 1)
        a_r, b_r = av[c % 2], bv[c % 2]
        w0_r, w1_r = w0v[c % 2], w1v[c % 2]

        def row_body(r, _):
            w0 = w0_r[r, pl.ds(0, _V)]
            w1 = w1_r[r, pl.ds(0, _V)]
            for j in range(_H // _V):
                s = pl.ds(j * _V, _V)
                a_r[r, s] = w0 * a_r[r, s] + w1 * b_r[r, s]
            return 0

        lax.fori_loop(0, _CC, row_body, 0)
        out_h[c] = pltpu.async_copy(
            a_r, out.at[pl.ds(base + c * _CC, _CC)], sem_out)
    out_h[_NCC - 2].wait()
    out_h[_NCC - 1].wait()


def _ffn_block(be_ref, x_ref, gu_ref, dp_ref, y_ref):
    x = x_ref[...].astype(jnp.float32)
    gu = gu_ref[0]
    g = lax.dot_general(x, gu, (((1,), (1,)), ((), ())),
                        preferred_element_type=jnp.float32)
    gate = g[:, :_I]
    up = g[:, _I:]
    h = up * (gate * jax.nn.sigmoid(gate))
    dp = dp_ref[0]
    y_ref[...] = lax.dot_general(h, dp, (((1,), (1,)), ((), ())),
                                 preferred_element_type=jnp.float32)


_ffn = pl.pallas_call(
    _ffn_block,
    grid_spec=pltpu.PrefetchScalarGridSpec(
        num_scalar_prefetch=1,
        grid=(_NB,),
        in_specs=[
            pl.BlockSpec((_B, _H), lambda b, be: (b, 0)),
            pl.BlockSpec((1, 2 * _I, _H), lambda b, be: (be[b], 0, 0)),
            pl.BlockSpec((1, _H, _I), lambda b, be: (be[b], 0, 0)),
        ],
        out_specs=pl.BlockSpec((_B, _H), lambda b, be: (b, 0)),
    ),
    out_shape=jax.ShapeDtypeStruct((_P, _H), jnp.float32),
    compiler_params=pltpu.CompilerParams(
        dimension_semantics=("arbitrary",),
    ),
)


def kernel(hidden_states, top_k_index, top_k_weights, gate_up_proj, down_proj):
    tki = top_k_index.astype(jnp.int32)
    eids = jnp.arange(_E, dtype=jnp.int32)
    oh0 = (tki[:, 0:1] == eids[None, :]).astype(jnp.int32)
    oh1 = (tki[:, 1:2] == eids[None, :]).astype(jnp.int32)
    both = oh0 + oh1
    s_incl = jnp.cumsum(both, axis=0)
    s_excl = s_incl - both
    counts = s_incl[-1]
    padded = ((counts + _B - 1) // _B) * _B
    ends = jnp.cumsum(padded)
    offsets = ends - padded
    rank0 = jnp.sum(s_excl * oh0, axis=1)
    rank1 = jnp.sum(s_excl * oh1, axis=1) + (tki[:, 0] == tki[:, 1])
    pos0 = jnp.sum(offsets[None, :] * oh0, axis=1) + rank0
    pos1 = jnp.sum(offsets[None, :] * oh1, axis=1) + rank1
    block_expert = jnp.minimum(
        jnp.sum((jnp.arange(_NB, dtype=jnp.int32)[:, None] * _B >=
                 ends[None, :]).astype(jnp.int32), axis=1),
        _E - 1)

    w = top_k_weights.astype(jnp.float32)
    w0m = jnp.broadcast_to(w[:, 0:1], (_T, _V))
    w1m = jnp.broadcast_to(w[:, 1:2], (_T, _V))

    hidden_i32 = jax.lax.bitcast_convert_type(
        hidden_states.astype(jnp.bfloat16).reshape(_T, _H // 2, 2), jnp.int32)
    x_sorted = jax.lax.bitcast_convert_type(
        _permute_rows(hidden_i32, pos0, pos1), jnp.bfloat16).reshape(_P, _H)
    y_sorted = _ffn(block_expert, x_sorted, gate_up_proj, down_proj)
    return _combine_rows(y_sorted, pos0, pos1, w0m, w1m)


# speedup vs baseline: 2.0772x; 2.0772x over previous
import functools

import jax
import jax.numpy as jnp
from jax import lax
from jax.experimental import pallas as pl
from jax.experimental.pallas import tpu as pltpu
from jax.experimental.pallas import tpu_sc as plsc

_E = 16
_K = 2
_H = 1024
_I = 512
_T = 4096
_B = 128
_P = 10240
_NB = _P // _B
_NC = 2
_NS = 16
_NW = _NC * _NS
_TW = _T // _NW
_GC = 32
_NGC = _TW // _GC
_CC = 16
_NCC = _TW // _CC
_V = 16

_MESH = dict(core_axis_name="c", subcore_axis_name="s")


@functools.partial(
    pl.kernel,
    out_type=jax.ShapeDtypeStruct((_P, _H), jnp.float32),
    mesh=plsc.VectorSubcoreMesh(**_MESH),
    scratch_types=[
        pltpu.VMEM((_GC,), jnp.int32),
        pltpu.VMEM((_GC,), jnp.int32),
        pltpu.VMEM((_GC,), jnp.int32),
        pltpu.VMEM((_GC,), jnp.int32),
        pltpu.VMEM((_GC, _H), jnp.float32),
        pltpu.VMEM((_GC, _H), jnp.float32),
        pltpu.SemaphoreType.DMA,
        pltpu.SemaphoreType.DMA,
    ],
)
def _permute_rows(src, pos0, pos1, out, i0a, i0b, i1a, i1b, bufa, bufb,
                  sem_in, sem_out):
    wid = lax.axis_index("s") * _NC + lax.axis_index("c")
    base = wid * _TW
    i0 = (i0a, i0b)
    i1 = (i1a, i1b)
    buf = (bufa, bufb)
    in_h = [None] * _NGC
    out_h = [None] * _NGC

    def load_idx(c):
        o = base + c * _GC
        pltpu.sync_copy(pos0.at[pl.ds(o, _GC)], i0[c % 2])
        pltpu.sync_copy(pos1.at[pl.ds(o, _GC)], i1[c % 2])

    load_idx(0)
    in_h[0] = pltpu.async_copy(src.at[pl.ds(base, _GC)], buf[0], sem_in)
    for c in range(_NGC):
        in_h[c].wait()
        if c + 1 < _NGC:
            if c >= 1:
                for h in out_h[c - 1]:
                    h.wait()
            load_idx(c + 1)
            o = base + (c + 1) * _GC
            in_h[c + 1] = pltpu.async_copy(
                src.at[pl.ds(o, _GC)], buf[(c + 1) % 2], sem_in)
        out_h[c] = (
            pltpu.async_copy(buf[c % 2], out.at[i0[c % 2]], sem_out),
            pltpu.async_copy(buf[c % 2], out.at[i1[c % 2]], sem_out),
        )
    for c in (_NGC - 2, _NGC - 1):
        for h in out_h[c]:
            h.wait()


@functools.partial(
    pl.kernel,
    out_type=jax.ShapeDtypeStruct((_T, _H), jnp.float32),
    mesh=plsc.VectorSubcoreMesh(**_MESH),
    scratch_types=[
        pltpu.VMEM((_CC,), jnp.int32),
        pltpu.VMEM((_CC,), jnp.int32),
        pltpu.VMEM((_CC,), jnp.int32),
        pltpu.VMEM((_CC,), jnp.int32),
        pltpu.VMEM((_CC, _V), jnp.float32),
        pltpu.VMEM((_CC, _V), jnp.float32),
        pltpu.VMEM((_CC, _V), jnp.float32),
        pltpu.VMEM((_CC, _V), jnp.float32),
        pltpu.VMEM((_CC, _H // 2), jnp.int32),
        pltpu.VMEM((_CC, _H // 2), jnp.int32),
        pltpu.VMEM((_CC, _H // 2), jnp.int32),
        pltpu.VMEM((_CC, _H // 2), jnp.int32),
        pltpu.VMEM((_CC, _H), jnp.float32),
        pltpu.VMEM((_CC, _H), jnp.float32),
        pltpu.SemaphoreType.DMA,
        pltpu.SemaphoreType.DMA,
    ],
)
def _combine_rows(y, pos0, pos1, w0m, w1m, out, i0a, i0b, i1a, i1b,
                  w0a, w0b, w1a, w1b, a0, a1, b0, b1, acc0, acc1,
                  sem_in, sem_out):
    wid = lax.axis_index("s") * _NC + lax.axis_index("c")
    base = wid * _TW
    i0 = (i0a, i0b)
    i1 = (i1a, i1b)
    w0v = (w0a, w0b)
    w1v = (w1a, w1b)
    av = (a0, a1)
    bv = (b0, b1)
    accv = (acc0, acc1)
    in_h = [None] * _NCC
    out_h = [None] * _NCC

    def start_chunk(c):
        o = base + c * _CC
        pltpu.sync_copy(pos0.at[pl.ds(o, _CC)], i0[c % 2])
        pltpu.sync_copy(pos1.at[pl.ds(o, _CC)], i1[c % 2])
        pltpu.sync_copy(w0m.at[pl.ds(o, _CC)], w0v[c % 2])
        pltpu.sync_copy(w1m.at[pl.ds(o, _CC)], w1v[c % 2])
        in_h[c] = (
            pltpu.async_copy(y.at[i0[c % 2]], av[c % 2], sem_in),
            pltpu.async_copy(y.at[i1[c % 2]], bv[c % 2], sem_in),
        )

    start_chunk(0)
    for c in range(_NCC):
        for h in in_h[c]:
            h.wait()
        if c + 1 < _NCC:
            if c >= 1:
                out_h[c - 1].wait()
            start_chunk(c + 1)
        a_r, b_r, acc_r = av[c % 2], bv[c % 2], accv[c % 2]
        w0_r, w1_r = w0v[c % 2], w1v[c % 2]
        himask = jnp.int32(-65536)

        def row_body(r, _):
            w0 = w0_r[r, pl.ds(0, _V)]
            w1 = w1_r[r, pl.ds(0, _V)]
            for j in range(_H // 2 // _V):
                s = pl.ds(j * _V, _V)
                wa = a_r[r, s]
                wb = b_r[r, s]
                lo_a = lax.bitcast_convert_type(wa << 16, jnp.float32)
                hi_a = lax.bitcast_convert_type(wa & himask, jnp.float32)
                lo_b = lax.bitcast_convert_type(wb << 16, jnp.float32)
                hi_b = lax.bitcast_convert_type(wb & himask, jnp.float32)
                acc_r[r, s] = w0 * lo_a + w1 * lo_b
                acc_r[r, pl.ds(_H // 2 + j * _V, _V)] = w0 * hi_a + w1 * hi_b
            return 0

        lax.fori_loop(0, _CC, row_body, 0)
        out_h[c] = pltpu.async_copy(
            acc_r, out.at[pl.ds(base + c * _CC, _CC)], sem_out)
    out_h[_NCC - 2].wait()
    out_h[_NCC - 1].wait()


def _rne_hi16(bits):
    return bits + jnp.int32(0x7FFF) + ((bits >> 16) & 1)


def _ffn_block(be_ref, x_ref, gu_ref, dp_ref, y_ref):
    x = x_ref[...]
    gu = gu_ref[0]
    g = lax.dot_general(x, gu, (((1,), (1,)), ((), ())),
                        preferred_element_type=jnp.float32)
    gate = g[:, :_I]
    up = g[:, _I:]
    h = up * (gate * jax.nn.sigmoid(gate))
    dp = dp_ref[0]
    y = lax.dot_general(h, dp, (((1,), (1,)), ((), ())),
                        preferred_element_type=jnp.float32)
    lo = jax.lax.bitcast_convert_type(y[:, :_H // 2], jnp.int32)
    hi = jax.lax.bitcast_convert_type(y[:, _H // 2:], jnp.int32)
    y_ref[...] = ((_rne_hi16(lo) >> 16) & jnp.int32(0xFFFF)) | (
        _rne_hi16(hi) & jnp.int32(-65536))


_ffn = pl.pallas_call(
    _ffn_block,
    grid_spec=pltpu.PrefetchScalarGridSpec(
        num_scalar_prefetch=1,
        grid=(_NB,),
        in_specs=[
            pl.BlockSpec((_B, _H), lambda b, be: (b, 0)),
            pl.BlockSpec((1, 2 * _I, _H), lambda b, be: (be[b], 0, 0)),
            pl.BlockSpec((1, _H, _I), lambda b, be: (be[b], 0, 0)),
        ],
        out_specs=pl.BlockSpec((_B, _H // 2), lambda b, be: (b, 0)),
    ),
    out_shape=jax.ShapeDtypeStruct((_P, _H // 2), jnp.int32),
    compiler_params=pltpu.CompilerParams(
        dimension_semantics=("arbitrary",),
    ),
)


def kernel(hidden_states, top_k_index, top_k_weights, gate_up_proj, down_proj):
    tki = top_k_index.astype(jnp.int32)
    eids = jnp.arange(_E, dtype=jnp.int32)
    oh0 = (tki[:, 0:1] == eids[None, :]).astype(jnp.int32)
    oh1 = (tki[:, 1:2] == eids[None, :]).astype(jnp.int32)
    both = oh0 + oh1
    s_incl = jnp.cumsum(both, axis=0)
    s_excl = s_incl - both
    counts = s_incl[-1]
    padded = ((counts + _B - 1) // _B) * _B
    ends = jnp.cumsum(padded)
    offsets = ends - padded
    rank0 = jnp.sum(s_excl * oh0, axis=1)
    rank1 = jnp.sum(s_excl * oh1, axis=1) + (tki[:, 0] == tki[:, 1])
    pos0 = jnp.sum(offsets[None, :] * oh0, axis=1) + rank0
    pos1 = jnp.sum(offsets[None, :] * oh1, axis=1) + rank1
    block_expert = jnp.minimum(
        jnp.sum((jnp.arange(_NB, dtype=jnp.int32)[:, None] * _B >=
                 ends[None, :]).astype(jnp.int32), axis=1),
        _E - 1)

    w = top_k_weights.astype(jnp.float32)
    w0m = jnp.broadcast_to(w[:, 0:1], (_T, _V))
    w1m = jnp.broadcast_to(w[:, 1:2], (_T, _V))

    x_sorted = _permute_rows(hidden_states, pos0, pos1)
    y_sorted = _ffn(block_expert, x_sorted, gate_up_proj, down_proj)
    return _combine_rows(y_sorted, pos0, pos1, w0m, w1m)


# speedup vs baseline: 2.3836x; 1.1475x over previous
import functools

import jax
import jax.numpy as jnp
from jax import lax
from jax.experimental import pallas as pl
from jax.experimental.pallas import tpu as pltpu
from jax.experimental.pallas import tpu_sc as plsc

_E = 16
_K = 2
_H = 1024
_I = 512
_T = 4096
_B = 128
_P = 10240
_NB = _P // _B
_NC = 2
_NS = 16
_NW = _NC * _NS
_TW = _T // _NW
_GC = 32
_NGC = _TW // _GC
_CC = 16
_NCC = _TW // _CC
_V = 16

_MESH = dict(core_axis_name="c", subcore_axis_name="s")


@functools.partial(
    pl.kernel,
    out_type=jax.ShapeDtypeStruct((_P, _H), jnp.float32),
    mesh=plsc.VectorSubcoreMesh(**_MESH),
    scratch_types=[
        pltpu.VMEM((_GC,), jnp.int32),
        pltpu.VMEM((_GC,), jnp.int32),
        pltpu.VMEM((_GC,), jnp.int32),
        pltpu.VMEM((_GC,), jnp.int32),
        pltpu.VMEM((_GC, _H), jnp.float32),
        pltpu.VMEM((_GC, _H), jnp.float32),
        pltpu.SemaphoreType.DMA,
        pltpu.SemaphoreType.DMA,
    ],
)
def _permute_rows(src, pos0, pos1, out, i0a, i0b, i1a, i1b, bufa, bufb,
                  sem_in, sem_out):
    wid = lax.axis_index("s") * _NC + lax.axis_index("c")
    base = wid * _TW
    i0 = (i0a, i0b)
    i1 = (i1a, i1b)
    buf = (bufa, bufb)
    in_h = [None] * _NGC
    out_h = [None] * _NGC

    def load_idx(c):
        o = base + c * _GC
        pltpu.sync_copy(pos0.at[pl.ds(o, _GC)], i0[c % 2])
        pltpu.sync_copy(pos1.at[pl.ds(o, _GC)], i1[c % 2])

    load_idx(0)
    in_h[0] = pltpu.async_copy(src.at[pl.ds(base, _GC)], buf[0], sem_in)
    for c in range(_NGC):
        in_h[c].wait()
        if c + 1 < _NGC:
            if c >= 1:
                for h in out_h[c - 1]:
                    h.wait()
            load_idx(c + 1)
            o = base + (c + 1) * _GC
            in_h[c + 1] = pltpu.async_copy(
                src.at[pl.ds(o, _GC)], buf[(c + 1) % 2], sem_in)
        out_h[c] = (
            pltpu.async_copy(buf[c % 2], out.at[i0[c % 2]], sem_out),
            pltpu.async_copy(buf[c % 2], out.at[i1[c % 2]], sem_out),
        )
    for c in (_NGC - 2, _NGC - 1):
        for h in out_h[c]:
            h.wait()


@functools.partial(
    pl.kernel,
    out_type=jax.ShapeDtypeStruct((_T, _H), jnp.float32),
    mesh=plsc.VectorSubcoreMesh(**_MESH),
    scratch_types=[
        pltpu.VMEM((_CC,), jnp.int32),
        pltpu.VMEM((_CC,), jnp.int32),
        pltpu.VMEM((_CC,), jnp.int32),
        pltpu.VMEM((_CC,), jnp.int32),
        pltpu.VMEM((_CC, _V), jnp.float32),
        pltpu.VMEM((_CC, _V), jnp.float32),
        pltpu.VMEM((_CC, _V), jnp.float32),
        pltpu.VMEM((_CC, _V), jnp.float32),
        pltpu.VMEM((_CC, _H), jnp.float32),
        pltpu.VMEM((_CC, _H), jnp.float32),
        pltpu.VMEM((_CC, _H), jnp.float32),
        pltpu.VMEM((_CC, _H), jnp.float32),
        pltpu.SemaphoreType.DMA,
        pltpu.SemaphoreType.DMA,
    ],
)
def _combine_rows(y, pos0, pos1, w0m, w1m, out, i0a, i0b, i1a, i1b,
                  w0a, w0b, w1a, w1b, a0, a1, b0, b1, sem_in, sem_out):
    wid = lax.axis_index("s") * _NC + lax.axis_index("c")
    base = wid * _TW
    i0 = (i0a, i0b)
    i1 = (i1a, i1b)
    w0v = (w0a, w0b)
    w1v = (w1a, w1b)
    av = (a0, a1)
    bv = (b0, b1)
    in_h = [None] * _NCC
    out_h = [None] * _NCC

    def start_chunk(c):
        o = base + c * _CC
        pltpu.sync_copy(pos0.at[pl.ds(o, _CC)], i0[c % 2])
        pltpu.sync_copy(pos1.at[pl.ds(o, _CC)], i1[c % 2])
        pltpu.sync_copy(w0m.at[pl.ds(o, _CC)], w0v[c % 2])
        pltpu.sync_copy(w1m.at[pl.ds(o, _CC)], w1v[c % 2])
        in_h[c] = (
            pltpu.async_copy(y.at[i0[c % 2]], av[c % 2], sem_in),
            pltpu.async_copy(y.at[i1[c % 2]], bv[c % 2], sem_in),
        )

    start_chunk(0)
    for c in range(_NCC):
        for h in in_h[c]:
            h.wait()
        if c + 1 < _NCC:
            if c >= 1:
                out_h[c - 1].wait()
            start_chunk(c + 1)
        a_r, b_r = av[c % 2], bv[c % 2]
        w0_r, w1_r = w0v[c % 2], w1v[c % 2]

        def row_body(r, _):
            w0 = w0_r[r, pl.ds(0, _V)]
            w1 = w1_r[r, pl.ds(0, _V)]
            for j in range(_H // _V):
                s = pl.ds(j * _V, _V)
                a_r[r, s] = w0 * a_r[r, s] + w1 * b_r[r, s]
            return 0

        lax.fori_loop(0, _CC, row_body, 0)
        out_h[c] = pltpu.async_copy(
            a_r, out.at[pl.ds(base + c * _CC, _CC)], sem_out)
    out_h[_NCC - 2].wait()
    out_h[_NCC - 1].wait()


def _ffn_block(meta_ref, x_ref, gu_hbm, dp_hbm, y_ref, gu_buf, dp_buf, sems):
    b = pl.program_id(0)
    par = meta_ref[0, b]
    first = meta_ref[1, b]
    nxt = meta_ref[2, b]
    hasn = meta_ref[3, b]
    cur = meta_ref[4, b]

    @pl.when(b == 0)
    def _start_first():
        pltpu.make_async_copy(gu_hbm.at[cur], gu_buf.at[par],
                              sems.at[par, 0]).start()
        pltpu.make_async_copy(dp_hbm.at[cur], dp_buf.at[par],
                              sems.at[par, 1]).start()

    @pl.when(first == 1)
    def _arrive_and_prefetch():
        pltpu.make_async_copy(gu_hbm.at[cur], gu_buf.at[par],
                              sems.at[par, 0]).wait()
        pltpu.make_async_copy(dp_hbm.at[cur], dp_buf.at[par],
                              sems.at[par, 1]).wait()

        @pl.when(hasn == 1)
        def _prefetch_next():
            pltpu.make_async_copy(gu_hbm.at[nxt], gu_buf.at[1 - par],
                                  sems.at[1 - par, 0]).start()
            pltpu.make_async_copy(dp_hbm.at[nxt], dp_buf.at[1 - par],
                                  sems.at[1 - par, 1]).start()

    x = x_ref[...]
    gu = gu_buf[par]
    g = lax.dot_general(x, gu, (((1,), (1,)), ((), ())),
                        preferred_element_type=jnp.float32)
    gate = g[:, :_I]
    up = g[:, _I:]
    h = up * (gate * jax.nn.sigmoid(gate))
    dp = dp_buf[par]
    y_ref[...] = lax.dot_general(h, dp, (((1,), (1,)), ((), ())),
                                 preferred_element_type=jnp.float32)


_ffn = pl.pallas_call(
    _ffn_block,
    grid_spec=pltpu.PrefetchScalarGridSpec(
        num_scalar_prefetch=1,
        grid=(_NB,),
        in_specs=[
            pl.BlockSpec((_B, _H), lambda b, meta: (b, 0)),
            pl.BlockSpec(memory_space=pltpu.MemorySpace.HBM),
            pl.BlockSpec(memory_space=pltpu.MemorySpace.HBM),
        ],
        out_specs=pl.BlockSpec((_B, _H), lambda b, meta: (b, 0)),
        scratch_shapes=[
            pltpu.VMEM((2, 2 * _I, _H), jnp.float32),
            pltpu.VMEM((2, _H, _I), jnp.float32),
            pltpu.SemaphoreType.DMA((2, 2)),
        ],
    ),
    out_shape=jax.ShapeDtypeStruct((_P, _H), jnp.float32),
    compiler_params=pltpu.CompilerParams(
        dimension_semantics=("arbitrary",),
    ),
)


def kernel(hidden_states, top_k_index, top_k_weights, gate_up_proj, down_proj):
    tki = top_k_index.astype(jnp.int32)
    eids = jnp.arange(_E, dtype=jnp.int32)
    oh0 = (tki[:, 0:1] == eids[None, :]).astype(jnp.int32)
    oh1 = (tki[:, 1:2] == eids[None, :]).astype(jnp.int32)
    both = oh0 + oh1
    s_incl = jnp.cumsum(both, axis=0)
    s_excl = s_incl - both
    counts = s_incl[-1]
    padded = ((counts + _B - 1) // _B) * _B
    ends = jnp.cumsum(padded)
    offsets = ends - padded
    rank0 = jnp.sum(s_excl * oh0, axis=1)
    rank1 = jnp.sum(s_excl * oh1, axis=1) + (tki[:, 0] == tki[:, 1])
    pos0 = jnp.sum(offsets[None, :] * oh0, axis=1) + rank0
    pos1 = jnp.sum(offsets[None, :] * oh1, axis=1) + rank1
    last_present = jnp.max(jnp.where(counts > 0, eids, 0))
    block_expert = jnp.minimum(
        jnp.sum((jnp.arange(_NB, dtype=jnp.int32)[:, None] * _B >=
                 ends[None, :]).astype(jnp.int32), axis=1),
        last_present)
    first = jnp.concatenate([
        jnp.ones((1,), jnp.int32),
        (block_expert[1:] != block_expert[:-1]).astype(jnp.int32)])
    parity = (jnp.cumsum(first) - 1) % 2
    big = jnp.int32(_E)
    cand = jnp.where((eids[None, :] > eids[:, None]) & (counts[None, :] > 0),
                     eids[None, :], big)
    next_present = jnp.min(cand, axis=1)
    oh_be = (block_expert[:, None] == eids[None, :]).astype(jnp.int32)
    nxt_e = jnp.sum(oh_be * next_present[None, :], axis=1)
    has_next = (nxt_e < big).astype(jnp.int32)
    meta = jnp.stack([parity.astype(jnp.int32), first,
                      jnp.minimum(nxt_e, _E - 1), has_next, block_expert])

    w = top_k_weights.astype(jnp.float32)
    w0m = jnp.broadcast_to(w[:, 0:1], (_T, _V))
    w1m = jnp.broadcast_to(w[:, 1:2], (_T, _V))

    x_sorted = _permute_rows(hidden_states, pos0, pos1)
    y_sorted = _ffn(meta, x_sorted, gate_up_proj, down_proj)
    return _combine_rows(y_sorted, pos0, pos1, w0m, w1m)


# speedup vs baseline: 2.4344x; 1.0213x over previous
import functools

import jax
import jax.numpy as jnp
from jax import lax
from jax.experimental import pallas as pl
from jax.experimental.pallas import tpu as pltpu
from jax.experimental.pallas import tpu_sc as plsc

_E = 16
_K = 2
_H = 1024
_I = 512
_T = 4096
_B = 128
_P = 10240
_NB = _P // _B
_NC = 2
_NS = 16
_NW = _NC * _NS
_TW = _T // _NW
_GC = 32
_NGC = _TW // _GC
_CC = 16
_NCC = _TW // _CC
_V = 16

_MESH = dict(core_axis_name="c", subcore_axis_name="s")


@functools.partial(
    pl.kernel,
    out_type=jax.ShapeDtypeStruct((_P, _H), jnp.float32),
    mesh=plsc.VectorSubcoreMesh(**_MESH),
    scratch_types=(
        [pltpu.VMEM((_GC,), jnp.int32)] * 6 +
        [pltpu.VMEM((_GC, _H), jnp.float32)] * 3 +
        [pltpu.SemaphoreType.DMA, pltpu.SemaphoreType.DMA]
    ),
)
def _permute_rows(src, pos0, pos1, out, i0a, i0b, i0c, i1a, i1b, i1c,
                  bufa, bufb, bufc, sem_in, sem_out):
    wid = lax.axis_index("s") * _NC + lax.axis_index("c")
    base = wid * _TW
    i0 = (i0a, i0b, i0c)
    i1 = (i1a, i1b, i1c)
    buf = (bufa, bufb, bufc)
    in_h = [None] * _NGC
    out_h = [None] * _NGC

    def start_chunk(c):
        o = base + c * _GC
        pltpu.sync_copy(pos0.at[pl.ds(o, _GC)], i0[c % 3])
        pltpu.sync_copy(pos1.at[pl.ds(o, _GC)], i1[c % 3])
        in_h[c] = pltpu.async_copy(src.at[pl.ds(o, _GC)], buf[c % 3], sem_in)

    start_chunk(0)
    start_chunk(1)
    for c in range(_NGC):
        in_h[c].wait()
        if c + 2 < _NGC:
            if c >= 1:
                for h in out_h[c - 1]:
                    h.wait()
            start_chunk(c + 2)
        out_h[c] = (
            pltpu.async_copy(buf[c % 3], out.at[i0[c % 3]], sem_out),
            pltpu.async_copy(buf[c % 3], out.at[i1[c % 3]], sem_out),
        )
    for c in range(max(0, _NGC - 3), _NGC):
        for h in out_h[c]:
            h.wait()


@functools.partial(
    pl.kernel,
    out_type=jax.ShapeDtypeStruct((_T, _H), jnp.float32),
    mesh=plsc.VectorSubcoreMesh(**_MESH),
    scratch_types=(
        [pltpu.VMEM((_CC,), jnp.int32)] * 6 +
        [pltpu.VMEM((_CC, _V), jnp.float32)] * 6 +
        [pltpu.VMEM((_CC, _H), jnp.float32)] * 6 +
        [pltpu.SemaphoreType.DMA, pltpu.SemaphoreType.DMA]
    ),
)
def _combine_rows(y, pos0, pos1, w0m, w1m, out, i0a, i0b, i0c, i1a, i1b, i1c,
                  w0a, w0b, w0c, w1a, w1b, w1c, a0, a1, a2, b0, b1, b2,
                  sem_in, sem_out):
    wid = lax.axis_index("s") * _NC + lax.axis_index("c")
    base = wid * _TW
    i0 = (i0a, i0b, i0c)
    i1 = (i1a, i1b, i1c)
    w0v = (w0a, w0b, w0c)
    w1v = (w1a, w1b, w1c)
    av = (a0, a1, a2)
    bv = (b0, b1, b2)
    in_h = [None] * _NCC
    out_h = [None] * _NCC

    def start_chunk(c):
        o = base + c * _CC
        pltpu.sync_copy(pos0.at[pl.ds(o, _CC)], i0[c % 3])
        pltpu.sync_copy(pos1.at[pl.ds(o, _CC)], i1[c % 3])
        pltpu.sync_copy(w0m.at[pl.ds(o, _CC)], w0v[c % 3])
        pltpu.sync_copy(w1m.at[pl.ds(o, _CC)], w1v[c % 3])
        in_h[c] = (
            pltpu.async_copy(y.at[i0[c % 3]], av[c % 3], sem_in),
            pltpu.async_copy(y.at[i1[c % 3]], bv[c % 3], sem_in),
        )

    start_chunk(0)
    start_chunk(1)
    for c in range(_NCC):
        for h in in_h[c]:
            h.wait()
        if c + 2 < _NCC:
            if c >= 1:
                out_h[c - 1].wait()
            start_chunk(c + 2)
        a_r, b_r = av[c % 3], bv[c % 3]
        w0_r, w1_r = w0v[c % 3], w1v[c % 3]

        def row_body(r, _):
            w0 = w0_r[r, pl.ds(0, _V)]
            w1 = w1_r[r, pl.ds(0, _V)]
            for j in range(_H // _V):
                s = pl.ds(j * _V, _V)
                a_r[r, s] = w0 * a_r[r, s] + w1 * b_r[r, s]
            return 0

        lax.fori_loop(0, _CC, row_body, 0)
        out_h[c] = pltpu.async_copy(
            a_r, out.at[pl.ds(base + c * _CC, _CC)], sem_out)
    for c in range(max(0, _NCC - 3), _NCC):
        out_h[c].wait()


def _ffn_block(meta_ref, x_ref, gu_hbm, dp_hbm, y_ref, gu_buf, dp_buf, sems):
    b = pl.program_id(0)
    par = meta_ref[0, b]
    first = meta_ref[1, b]
    nxt = meta_ref[2, b]
    hasn = meta_ref[3, b]
    cur = meta_ref[4, b]

    @pl.when(b == 0)
    def _start_first():
        pltpu.make_async_copy(gu_hbm.at[cur], gu_buf.at[par],
                              sems.at[par, 0]).start()
        pltpu.make_async_copy(dp_hbm.at[cur], dp_buf.at[par],
                              sems.at[par, 1]).start()

    @pl.when(first == 1)
    def _arrive_and_prefetch():
        pltpu.make_async_copy(gu_hbm.at[cur], gu_buf.at[par],
                              sems.at[par, 0]).wait()
        pltpu.make_async_copy(dp_hbm.at[cur], dp_buf.at[par],
                              sems.at[par, 1]).wait()

        @pl.when(hasn == 1)
        def _prefetch_next():
            pltpu.make_async_copy(gu_hbm.at[nxt], gu_buf.at[1 - par],
                                  sems.at[1 - par, 0]).start()
            pltpu.make_async_copy(dp_hbm.at[nxt], dp_buf.at[1 - par],
                                  sems.at[1 - par, 1]).start()

    x = x_ref[...]
    gu = gu_buf[par]
    g = lax.dot_general(x, gu, (((1,), (1,)), ((), ())),
                        preferred_element_type=jnp.float32)
    gate = g[:, :_I]
    up = g[:, _I:]
    h = up * (gate * jax.nn.sigmoid(gate))
    dp = dp_buf[par]
    y_ref[...] = lax.dot_general(h, dp, (((1,), (1,)), ((), ())),
                                 preferred_element_type=jnp.float32)


_ffn = pl.pallas_call(
    _ffn_block,
    grid_spec=pltpu.PrefetchScalarGridSpec(
        num_scalar_prefetch=1,
        grid=(_NB,),
        in_specs=[
            pl.BlockSpec((_B, _H), lambda b, meta: (b, 0)),
            pl.BlockSpec(memory_space=pltpu.MemorySpace.HBM),
            pl.BlockSpec(memory_space=pltpu.MemorySpace.HBM),
        ],
        out_specs=pl.BlockSpec((_B, _H), lambda b, meta: (b, 0)),
        scratch_shapes=[
            pltpu.VMEM((2, 2 * _I, _H), jnp.float32),
            pltpu.VMEM((2, _H, _I), jnp.float32),
            pltpu.SemaphoreType.DMA((2, 2)),
        ],
    ),
    out_shape=jax.ShapeDtypeStruct((_P, _H), jnp.float32),
    compiler_params=pltpu.CompilerParams(
        dimension_semantics=("arbitrary",),
    ),
)


def kernel(hidden_states, top_k_index, top_k_weights, gate_up_proj, down_proj):
    tki = top_k_index.astype(jnp.int32)
    eids = jnp.arange(_E, dtype=jnp.int32)
    oh0 = (tki[:, 0:1] == eids[None, :]).astype(jnp.int32)
    oh1 = (tki[:, 1:2] == eids[None, :]).astype(jnp.int32)
    both = oh0 + oh1
    s_incl = jnp.cumsum(both, axis=0)
    s_excl = s_incl - both
    counts = s_incl[-1]
    padded = ((counts + _B - 1) // _B) * _B
    ends = jnp.cumsum(padded)
    offsets = ends - padded
    rank0 = jnp.sum(s_excl * oh0, axis=1)
    rank1 = jnp.sum(s_excl * oh1, axis=1) + (tki[:, 0] == tki[:, 1])
    pos0 = jnp.sum(offsets[None, :] * oh0, axis=1) + rank0
    pos1 = jnp.sum(offsets[None, :] * oh1, axis=1) + rank1
    last_present = jnp.max(jnp.where(counts > 0, eids, 0))
    block_expert = jnp.minimum(
        jnp.sum((jnp.arange(_NB, dtype=jnp.int32)[:, None] * _B >=
                 ends[None, :]).astype(jnp.int32), axis=1),
        last_present)
    first = jnp.concatenate([
        jnp.ones((1,), jnp.int32),
        (block_expert[1:] != block_expert[:-1]).astype(jnp.int32)])
    parity = (jnp.cumsum(first) - 1) % 2
    big = jnp.int32(_E)
    cand = jnp.where((eids[None, :] > eids[:, None]) & (counts[None, :] > 0),
                     eids[None, :], big)
    next_present = jnp.min(cand, axis=1)
    oh_be = (block_expert[:, None] == eids[None, :]).astype(jnp.int32)
    nxt_e = jnp.sum(oh_be * next_present[None, :], axis=1)
    has_next = (nxt_e < big).astype(jnp.int32)
    meta = jnp.stack([parity.astype(jnp.int32), first,
                      jnp.minimum(nxt_e, _E - 1), has_next, block_expert])

    w = top_k_weights.astype(jnp.float32)
    w0m = jnp.broadcast_to(w[:, 0:1], (_T, _V))
    w1m = jnp.broadcast_to(w[:, 1:2], (_T, _V))

    x_sorted = _permute_rows(hidden_states, pos0, pos1)
    y_sorted = _ffn(meta, x_sorted, gate_up_proj, down_proj)
    return _combine_rows(y_sorted, pos0, pos1, w0m, w1m)
